# baseline (speedup 1.0000x reference)
import functools

import jax
import jax.numpy as jnp
from jax import lax
from jax.experimental import pallas as pl
from jax.experimental.pallas import tpu as pltpu
from jax.experimental.pallas import tpu_sc as plsc

EMB = 32
LANES = 16
CHUNK = 128


@functools.lru_cache(maxsize=None)
def _build(batch):
    nc, ns = 2, 16
    nw = nc * ns
    per_w = batch // nw
    n_chunks = per_w // CHUNK
    n_groups = per_w // LANES
    mesh = plsc.VectorSubcoreMesh(core_axis_name="c", subcore_axis_name="s")

    @functools.partial(
        pl.kernel,
        mesh=mesh,
        compiler_params=pltpu.CompilerParams(
            needs_layout_passes=False, use_tc_tiling_on_sc=False),
        out_type=jax.ShapeDtypeStruct((batch,), jnp.float32),
        scratch_types=[
            pltpu.VMEM((per_w,), jnp.int32),
            pltpu.VMEM((per_w,), jnp.int32),
            pltpu.VMEM((EMB * per_w,), jnp.float32),
            pltpu.VMEM((EMB * per_w,), jnp.float32),
            pltpu.VMEM((per_w,), jnp.float32),
            pltpu.VMEM((per_w,), jnp.float32),
            pltpu.VMEM((per_w,), jnp.float32),
            pltpu.SemaphoreType.DMA,
            pltpu.SemaphoreType.DMA,
        ],
    )
    def k(uembT, membT, ubias, mbias, uids, mids, out_hbm,
          uid_v, mid_v, ur, mr, ub_v, mb_v, out_v, sem, bsem):
        wid = lax.axis_index("s") * nc + lax.axis_index("c")
        base = wid * per_w
        pltpu.sync_copy(uids.at[pl.ds(base, per_w)], uid_v)
        pltpu.sync_copy(mids.at[pl.ds(base, per_w)], mid_v)

        bias_copies = []
        for c in range(n_chunks):
            s = pl.ds(c * CHUNK, CHUNK)
            bias_copies.append(pltpu.async_copy(
                ubias.at[uid_v.at[s]], ub_v.at[s], bsem))
            bias_copies.append(pltpu.async_copy(
                mbias.at[mid_v.at[s]], mb_v.at[s], bsem))
            for d in range(EMB):
                o = pl.ds(d * per_w + c * CHUNK, CHUNK)
                pltpu.make_async_copy(
                    uembT.at[d].at[uid_v.at[s]], ur.at[o], sem).start()
                pltpu.make_async_copy(
                    membT.at[d].at[mid_v.at[s]], mr.at[o], sem).start()

        pltpu.make_async_copy(
            uembT.at[0].at[pl.ds(0, EMB * per_w)], ur, sem).wait()
        pltpu.make_async_copy(
            membT.at[0].at[pl.ds(0, EMB * per_w)], mr, sem).wait()
        for cp in bias_copies:
            cp.wait()

        def group(g, carry):
            j0 = g * LANES
            acc = ub_v[pl.ds(j0, LANES)] + mb_v[pl.ds(j0, LANES)]
            for d in range(EMB):
                acc = acc + (ur[pl.ds(d * per_w + j0, LANES)]
                             * mr[pl.ds(d * per_w + j0, LANES)])
            out_v[pl.ds(j0, LANES)] = acc
            return carry

        lax.fori_loop(0, n_groups, group, 0)
        pltpu.sync_copy(out_v, out_hbm.at[pl.ds(base, per_w)])

    return k


def kernel(user_ids, movie_ids, user_emb, movie_emb, user_bias, movie_bias):
    batch = user_ids.shape[0]
    k = _build(batch)
    return k(user_emb.T, movie_emb.T,
             user_bias.reshape(-1), movie_bias.reshape(-1),
             user_ids.astype(jnp.int32), movie_ids.astype(jnp.int32))


# speedup vs baseline: 4.6311x; 4.6311x over previous
import functools

import jax
import jax.numpy as jnp
from jax import lax
from jax.experimental import pallas as pl
from jax.experimental.pallas import tpu as pltpu
from jax.experimental.pallas import tpu_sc as plsc

EMB = 32
CHUNK = 128


@functools.lru_cache(maxsize=None)
def _build_gather(batch):
    nc, ns = 2, 16
    nw = nc * ns
    per_w = batch // nw
    n_chunks = per_w // CHUNK
    mesh = plsc.VectorSubcoreMesh(core_axis_name="c", subcore_axis_name="s")

    @functools.partial(
        pl.kernel,
        mesh=mesh,
        compiler_params=pltpu.CompilerParams(
            needs_layout_passes=False, use_tc_tiling_on_sc=False),
        out_type=(
            jax.ShapeDtypeStruct((batch, EMB), jnp.float32),
            jax.ShapeDtypeStruct((batch, EMB), jnp.float32),
            jax.ShapeDtypeStruct((batch,), jnp.float32),
            jax.ShapeDtypeStruct((batch,), jnp.float32),
        ),
        scratch_types=[
            pltpu.VMEM((per_w,), jnp.int32),
            pltpu.VMEM((per_w,), jnp.int32),
            pltpu.VMEM((per_w, EMB), jnp.float32),
            pltpu.VMEM((per_w, EMB), jnp.float32),
            pltpu.VMEM((per_w,), jnp.float32),
            pltpu.VMEM((per_w,), jnp.float32),
            pltpu.SemaphoreType.DMA,
        ],
    )
    def k(uemb, memb, ubias, mbias, uids, mids,
          ue_out, me_out, ub_out, mb_out,
          uid_v, mid_v, ur, mr, ub_v, mb_v, sem):
        wid = lax.axis_index("s") * nc + lax.axis_index("c")
        base = wid * per_w
        pltpu.sync_copy(uids.at[pl.ds(base, per_w)], uid_v)
        pltpu.sync_copy(mids.at[pl.ds(base, per_w)], mid_v)

        copies = []
        for c in range(n_chunks):
            s = pl.ds(c * CHUNK, CHUNK)
            copies.append(pltpu.async_copy(
                uemb.at[uid_v.at[s]], ur.at[s], sem))
            copies.append(pltpu.async_copy(
                memb.at[mid_v.at[s]], mr.at[s], sem))
            copies.append(pltpu.async_copy(
                ubias.at[uid_v.at[s]], ub_v.at[s], sem))
            copies.append(pltpu.async_copy(
                mbias.at[mid_v.at[s]], mb_v.at[s], sem))
        for cp in copies:
            cp.wait()

        pltpu.sync_copy(ur, ue_out.at[pl.ds(base, per_w)])
        pltpu.sync_copy(mr, me_out.at[pl.ds(base, per_w)])
        pltpu.sync_copy(ub_v, ub_out.at[pl.ds(base, per_w)])
        pltpu.sync_copy(mb_v, mb_out.at[pl.ds(base, per_w)])

    return k


def _dot_kernel(ue, me, ub, mb, out):
    out[...] = jnp.sum(ue[...] * me[...], axis=-1) + ub[...] + mb[...]


@functools.lru_cache(maxsize=None)
def _build_dot(batch):
    return pl.pallas_call(
        _dot_kernel,
        out_shape=jax.ShapeDtypeStruct((batch,), jnp.float32),
    )


def kernel(user_ids, movie_ids, user_emb, movie_emb, user_bias, movie_bias):
    batch = user_ids.shape[0]
    ue, me, ub, mb = _build_gather(batch)(
        user_emb, movie_emb,
        user_bias.reshape(-1), movie_bias.reshape(-1),
        user_ids.astype(jnp.int32), movie_ids.astype(jnp.int32))
    return _build_dot(batch)(ue, me, ub, mb)


# speedup vs baseline: 4.6350x; 1.0008x over previous
import functools

import jax
import jax.numpy as jnp
from jax import lax
from jax.experimental import pallas as pl
from jax.experimental.pallas import tpu as pltpu
from jax.experimental.pallas import tpu_sc as plsc

EMB = 32
CHUNK = 128


@functools.lru_cache(maxsize=None)
def _build_gather(batch):
    nc, ns = 2, 16
    nw = nc * ns
    per_w = batch // nw
    n_chunks = per_w // CHUNK
    mesh = plsc.VectorSubcoreMesh(core_axis_name="c", subcore_axis_name="s")

    @functools.partial(
        pl.kernel,
        mesh=mesh,
        compiler_params=pltpu.CompilerParams(
            needs_layout_passes=True, use_tc_tiling_on_sc=False),
        out_type=(
            jax.ShapeDtypeStruct((batch, EMB), jnp.float32),
            jax.ShapeDtypeStruct((batch, EMB), jnp.float32),
            jax.ShapeDtypeStruct((batch,), jnp.float32),
            jax.ShapeDtypeStruct((batch,), jnp.float32),
        ),
        scratch_types=[
            pltpu.VMEM((per_w,), jnp.int32),
            pltpu.VMEM((per_w,), jnp.int32),
            pltpu.VMEM((per_w, EMB), jnp.float32),
            pltpu.VMEM((per_w, EMB), jnp.float32),
            pltpu.VMEM((per_w,), jnp.float32),
            pltpu.VMEM((per_w,), jnp.float32),
            pltpu.SemaphoreType.DMA,
        ],
    )
    def k(uemb, memb, ubias, mbias, uids, mids,
          ue_out, me_out, ub_out, mb_out,
          uid_v, mid_v, ur, mr, ub_v, mb_v, sem):
        wid = lax.axis_index("s") * nc + lax.axis_index("c")
        base = wid * per_w
        pltpu.sync_copy(uids.at[pl.ds(base, per_w)], uid_v)
        pltpu.sync_copy(mids.at[pl.ds(base, per_w)], mid_v)

        copies = []
        for c in range(n_chunks):
            s = pl.ds(c * CHUNK, CHUNK)
            copies.append(pltpu.async_copy(
                uemb.at[uid_v.at[s]], ur.at[s], sem))
            copies.append(pltpu.async_copy(
                memb.at[mid_v.at[s]], mr.at[s], sem))
            copies.append(pltpu.async_copy(
                ubias.at[uid_v.at[s]], ub_v.at[s], sem))
            copies.append(pltpu.async_copy(
                mbias.at[mid_v.at[s]], mb_v.at[s], sem))
        for cp in copies:
            cp.wait()

        pltpu.sync_copy(ur, ue_out.at[pl.ds(base, per_w)])
        pltpu.sync_copy(mr, me_out.at[pl.ds(base, per_w)])
        pltpu.sync_copy(ub_v, ub_out.at[pl.ds(base, per_w)])
        pltpu.sync_copy(mb_v, mb_out.at[pl.ds(base, per_w)])

    return k


def _dot_kernel(ue, me, ub, mb, out):
    out[...] = jnp.sum(ue[...] * me[...], axis=-1) + ub[...] + mb[...]


@functools.lru_cache(maxsize=None)
def _build_dot(batch):
    return pl.pallas_call(
        _dot_kernel,
        out_shape=jax.ShapeDtypeStruct((batch,), jnp.float32),
    )


def kernel(user_ids, movie_ids, user_emb, movie_emb, user_bias, movie_bias):
    batch = user_ids.shape[0]
    ue, me, ub, mb = _build_gather(batch)(
        user_emb, movie_emb,
        user_bias.reshape(-1), movie_bias.reshape(-1),
        user_ids.astype(jnp.int32), movie_ids.astype(jnp.int32))
    return _build_dot(batch)(ue, me, ub, mb)
